# baseline (speedup 1.0000x reference)
import jax
import jax.numpy as jnp
from jax.experimental import pallas as pl
from jax.experimental.pallas import tpu as pltpu

_CH = 512


def _make_body(B, S, H, NC):
    def body(pos_hbm, out_hbm, vbuf, in_sem, out_sem):
        def in_copy(i, slot):
            return pltpu.make_async_copy(
                pos_hbm.at[pl.ds(i * _CH, _CH), :], vbuf.at[slot], in_sem.at[slot]
            )

        def out_copy(i, slot, b):
            return pltpu.make_async_copy(
                vbuf.at[slot],
                out_hbm.at[b, pl.ds(i * _CH, _CH), :],
                out_sem.at[slot, b],
            )

        in_copy(0, 0).start()
        for i in range(NC):
            slot = i % 2
            in_copy(i, slot).wait()
            if i + 1 < NC:
                if i >= 1:
                    for b in range(B):
                        out_copy(i - 1, 1 - slot, b).wait()
                in_copy(i + 1, 1 - slot).start()
            for b in range(B):
                out_copy(i, slot, b).start()
        for i in (NC - 2, NC - 1):
            for b in range(B):
                out_copy(i, i % 2, b).wait()

    return body


def kernel(x, pos_emb):
    B, S = x.shape
    N, H = pos_emb.shape
    NC = S // _CH
    out = pl.pallas_call(
        _make_body(B, S, H, NC),
        in_specs=[pl.BlockSpec(memory_space=pltpu.MemorySpace.HBM)],
        out_specs=pl.BlockSpec(memory_space=pltpu.MemorySpace.HBM),
        out_shape=jax.ShapeDtypeStruct((B, S, H), pos_emb.dtype),
        scratch_shapes=[
            pltpu.VMEM((2, _CH, H), pos_emb.dtype),
            pltpu.SemaphoreType.DMA((2,)),
            pltpu.SemaphoreType.DMA((2, 4)),
        ],
    )(pos_emb)
    return out


# speedup vs baseline: 1.1711x; 1.1711x over previous
import jax
import jax.numpy as jnp
from jax.experimental import pallas as pl

_BS = 1024


def _copy_kernel(pos_ref, out_ref):
    blk = pos_ref[...]
    out_ref[...] = jnp.broadcast_to(blk[None, :, :], out_ref.shape)


def kernel(x, pos_emb):
    B, S = x.shape
    N, H = pos_emb.shape
    grid = (S // _BS,)
    out = pl.pallas_call(
        _copy_kernel,
        grid=grid,
        in_specs=[pl.BlockSpec((_BS, H), lambda j: (j, 0))],
        out_specs=pl.BlockSpec((B, _BS, H), lambda j: (0, j, 0)),
        out_shape=jax.ShapeDtypeStruct((B, S, H), pos_emb.dtype),
    )(pos_emb)
    return out


# speedup vs baseline: 1.1746x; 1.0030x over previous
import jax
import jax.numpy as jnp
from jax.experimental import pallas as pl
from jax.experimental.pallas import tpu as pltpu

_BS = 1024


def _copy_kernel(pos_ref, out_ref):
    blk = pos_ref[...]
    out_ref[...] = jnp.broadcast_to(blk[None, :, :], out_ref.shape)


def kernel(x, pos_emb):
    B, S = x.shape
    N, H = pos_emb.shape
    grid = (S // _BS,)
    out = pl.pallas_call(
        _copy_kernel,
        grid=grid,
        in_specs=[pl.BlockSpec((_BS, H), lambda j: (j, 0))],
        out_specs=pl.BlockSpec((B, _BS, H), lambda j: (0, j, 0)),
        out_shape=jax.ShapeDtypeStruct((B, S, H), pos_emb.dtype),
        compiler_params=pltpu.CompilerParams(vmem_limit_bytes=128 * 1024 * 1024),
    )(pos_emb)
    return out
